# baseline (speedup 1.0000x reference)
import functools

import jax
import jax.numpy as jnp
from jax import lax
from jax.experimental import pallas as pl
from jax.experimental.pallas import tpu as pltpu
from jax.experimental.pallas import tpu_sc as plsc

N = 10000
D = 128
E = 320000
ALPHA = 0.7
BETA = 0.3
GAMMA = 0.2

NC = 2
NS = 16
NW = NC * NS
EPT = E // NW
ROWS = 80
RPAD = ROWS * 128
NB2 = 160
BE = 64

_mesh = plsc.VectorSubcoreMesh(core_axis_name="c", subcore_axis_name="s")


def _scores_body(wa_ref, wfc_ref, h_ref, out_ref):
    A = jnp.dot(wa_ref[...], wfc_ref[...], preferred_element_type=jnp.float32)
    s = lax.dot_general(
        A, h_ref[...], (((1,), (1,)), ((), ())),
        preferred_element_type=jnp.float32)
    out_ref[...] = jnp.pad(s, ((0, 0), (0, RPAD - N)))


def _scores(wa, wfc, h):
    return pl.pallas_call(
        _scores_body,
        out_shape=jax.ShapeDtypeStruct((2, RPAD), jnp.float32),
    )(wa, wfc, h)


def _edge_body(s_hbm, sd_hbm, p_hbm, dpart_hbm,
               s1_t, s2_t, sd_t, p_t, zb_t, dsum, sem):
    cid = lax.axis_index("c")
    sid = lax.axis_index("s")
    wid = sid * NC + cid

    pltpu.sync_copy(s_hbm.at[0], s1_t)
    pltpu.sync_copy(s_hbm.at[1], s2_t)
    pltpu.sync_copy(sd_hbm.at[wid], sd_t)

    for c in range(128):
        zb_t[pl.ds(c * 16, 16)] = jnp.zeros((16,), jnp.float32)

    @pl.when(sid == 0)
    def _():
        for k in range(RPAD // 2048):
            pltpu.sync_copy(zb_t, dsum.at[pl.ds(k * 2048, 2048)])

    plsc.subcore_barrier()

    iota16 = lax.broadcasted_iota(jnp.int32, (16,), 0)

    def row(r, _):
        def col(c, _):
            srcv = sd_t[r, 0, pl.ds(c * 16, 16)]
            dstv = sd_t[r, 1, pl.ds(c * 16, 16)]
            s1v = plsc.load_gather(s1_t, [srcv])
            s2v = plsc.load_gather(s2_t, [dstv])
            a = s1v + s2v
            e = jnp.where(a > 0, a, GAMMA * a)
            e = jnp.minimum(e, 60.0)
            p = jnp.exp(e)
            lid = r * 64 + c * 16 + iota16
            p = jnp.where(lid < EPT, p, 0.0)
            p_t[r, 0, pl.ds(c * 16, 16)] = p
            return 0
        lax.fori_loop(0, 4, col, 0)
        return 0
    lax.fori_loop(0, NB2, row, 0)

    def srow(r, _):
        pltpu.sync_copy(p_t.at[r, 0], dsum.at[sd_t.at[r, 1]], add=True)
        return 0
    lax.fori_loop(0, NB2, srow, 0)

    plsc.subcore_barrier()

    pltpu.sync_copy(p_t, p_hbm.at[wid])

    @pl.when(sid == 0)
    def _():
        pltpu.sync_copy(dsum, dpart_hbm.at[cid])


@functools.partial(
    pl.kernel,
    out_type=(jax.ShapeDtypeStruct((NW, NB2, 1, BE), jnp.float32),
              jax.ShapeDtypeStruct((NC, RPAD), jnp.float32)),
    mesh=_mesh,
    compiler_params=pltpu.CompilerParams(needs_layout_passes=False),
    scratch_types=[
        pltpu.VMEM((RPAD,), jnp.float32),
        pltpu.VMEM((RPAD,), jnp.float32),
        pltpu.VMEM((NB2, 2, BE), jnp.int32),
        pltpu.VMEM((NB2, 1, BE), jnp.float32),
        pltpu.VMEM((2048,), jnp.float32),
        pltpu.VMEM_SHARED((RPAD,), jnp.float32),
        pltpu.SemaphoreType.DMA,
    ],
)
def _edge_kernel(s_hbm, sd_hbm, p_hbm, dpart_hbm, *scratch):
    _edge_body(s_hbm, sd_hbm, p_hbm, dpart_hbm, *scratch)


_DNUMS = lax.GatherDimensionNumbers(
    offset_dims=(), collapsed_slice_dims=(0,), start_index_map=(0,))


def _msg_body(dpart_hbm, p_hbm, sd_hbm, h_hbm, acc_hbm,
              denom_t, dtmp_t, sd0, sd1, sd2, sd3, sd4, sd5, sd6, sd7,
              pr0, pr1, pr2, pr3, pr4, pr5, pr6, pr7,
              rb0, rb1, rb2, rb3, hsp, acc,
              si0, si1, si2, si3, si4, si5, si6, si7,
              sg0, sg1, sg2, sg3, ss0, ss1, ss2, ss3):
    cid = lax.axis_index("c")
    sid = lax.axis_index("s")
    wid = sid * NC + cid
    sd = (sd0, sd1, sd2, sd3, sd4, sd5, sd6, sd7)
    pr = (pr0, pr1, pr2, pr3, pr4, pr5, pr6, pr7)
    si = (si0, si1, si2, si3, si4, si5, si6, si7)
    rb = (rb0, rb1, rb2, rb3)
    sg = (sg0, sg1, sg2, sg3)
    ss = (ss0, ss1, ss2, ss3)
    base = sid * 640

    pltpu.sync_copy(dpart_hbm.at[0], denom_t)
    pltpu.sync_copy(dpart_hbm.at[1], dtmp_t)

    def dadd(i, _):
        denom_t[pl.ds(i * 16, 16)] = (denom_t[pl.ds(i * 16, 16)]
                                      + dtmp_t[pl.ds(i * 16, 16)])
        return 0
    lax.fori_loop(0, RPAD // 16, dadd, 0)

    def zrow(i, _):
        def zcol(c, _):
            rb0[i, pl.ds(c * 32, 32)] = jnp.zeros((32,), jnp.bfloat16)
            return 0
        lax.fori_loop(0, 4, zcol, 0)
        return 0
    lax.fori_loop(0, BE, zrow, 0)
    for k in range(10):
        pltpu.sync_copy(rb0, acc.at[pl.ds(base + k * BE, BE)])

    for k in range(10):
        pltpu.sync_copy(h_hbm.at[pl.ds(base + k * BE, BE)], rb1)
        pltpu.sync_copy(rb1, hsp.at[pl.ds(base + k * BE, BE)])

    plsc.subcore_barrier()

    def start_idx(r, s):
        pltpu.async_copy(sd_hbm.at[wid, r], sd[s], si[s])
        pltpu.async_copy(p_hbm.at[wid, r], pr[s], si[s])

    def wait_idx(r, s):
        pltpu.make_async_copy(sd_hbm.at[wid, r], sd[s], si[s]).wait()
        pltpu.make_async_copy(p_hbm.at[wid, r], pr[s], si[s]).wait()

    def start_gather(s, b):
        pltpu.async_copy(hsp.at[sd[s].at[0]], rb[b], sg[b])

    def wait_gather(b):
        pltpu.make_async_copy(hsp.at[sd[0].at[0]], rb[b], sg[b]).wait()

    def start_scatter(s, b):
        pltpu.async_copy(rb[b], acc.at[sd[s].at[1]], ss[b], add=True)

    def wait_scatter(b):
        pltpu.make_async_copy(rb[b], acc.at[sd[0].at[1]], ss[b]).wait()

    start_idx(0, 0)
    start_idx(1, 1)
    start_idx(2, 2)
    wait_idx(0, 0)
    start_gather(0, 0)
    wait_idx(1, 1)
    start_gather(1, 1)

    def oct_(i, _):
        for k in range(8):
            r = i * 8 + k
            b = k % 4
            s = k

            @pl.when(r + 3 < NB2)
            def _():
                start_idx(r + 3, (k + 3) % 8)

            @pl.when(r + 2 < NB2)
            def _():
                @pl.when(r >= 2)
                def _():
                    wait_scatter((k + 2) % 4)
                wait_idx(r + 2, (k + 2) % 8)
                start_gather((k + 2) % 8, (k + 2) % 4)

            wait_gather(b)

            def scale(c, _):
                dstv = sd[s][1, pl.ds(c * 16, 16)]
                dv = plsc.load_gather(denom_t, [dstv])
                pv = pr[s][0, pl.ds(c * 16, 16)]
                attnv = pv / (dv + 1e-16)
                for j in range(16):
                    sp = lax.gather(
                        attnv, jnp.full((16, 1), j, jnp.int32), _DNUMS, (1,),
                        mode=lax.GatherScatterMode.PROMISE_IN_BOUNDS)
                    spb = plsc.pack(sp, sp, format=plsc.PackFormat.INTERLEAVED)
                    row = c * 16 + j
                    for dch in range(4):
                        rb[b][row, pl.ds(dch * 32, 32)] = (
                            rb[b][row, pl.ds(dch * 32, 32)] * spb)
                return 0
            lax.fori_loop(0, BE // 16, scale, 0)

            start_scatter(s, b)
        return 0
    lax.fori_loop(0, NB2 // 8, oct_, 0)

    for b in range(4):
        wait_scatter(b)

    plsc.subcore_barrier()

    for k in range(5):
        pltpu.sync_copy(acc.at[pl.ds(base + k * 128, 128)],
                        acc_hbm.at[cid, pl.ds(base + k * 128, 128)])


@functools.partial(
    pl.kernel,
    out_type=jax.ShapeDtypeStruct((NC, RPAD, D), jnp.bfloat16),
    mesh=_mesh,
    compiler_params=pltpu.CompilerParams(
        needs_layout_passes=False, use_tc_tiling_on_sc=False),
    scratch_types=(
        [pltpu.VMEM((RPAD,), jnp.float32)]
        + [pltpu.VMEM((RPAD,), jnp.float32)]
        + [pltpu.VMEM((2, BE), jnp.int32)] * 8
        + [pltpu.VMEM((1, BE), jnp.float32)] * 8
        + [pltpu.VMEM((BE, D), jnp.bfloat16)] * 4
        + [pltpu.VMEM_SHARED((RPAD, D), jnp.bfloat16)]
        + [pltpu.VMEM_SHARED((RPAD, D), jnp.bfloat16)]
        + [pltpu.SemaphoreType.DMA] * 16
    ),
)
def _msg_kernel(dpart_hbm, p_hbm, sd_hbm, h_hbm, acc_hbm, *scratch):
    _msg_body(dpart_hbm, p_hbm, sd_hbm, h_hbm, acc_hbm, *scratch)


def _blend_body(emb_ref, acc_ref, out_ref):
    crf = (acc_ref[0].astype(jnp.float32) + acc_ref[1].astype(jnp.float32))
    out_ref[...] = (ALPHA * emb_ref[...] + BETA * crf) / (ALPHA + BETA)


def _blend(emb, acc):
    blk = 2000
    return pl.pallas_call(
        _blend_body,
        grid=(N // blk,),
        in_specs=[pl.BlockSpec((blk, D), lambda g: (g, 0)),
                  pl.BlockSpec((NC, blk, D), lambda g: (0, g, 0))],
        out_specs=pl.BlockSpec((blk, D), lambda g: (g, 0)),
        out_shape=jax.ShapeDtypeStruct((N, D), jnp.float32),
    )(emb, acc)


def kernel(embedding_input, h_input, edge_index, W_fc, W_attn):
    wa = W_attn.reshape(2, D)
    s = _scores(wa, W_fc, h_input)

    src = edge_index[0].reshape(NW, EPT)
    dst = edge_index[1].reshape(NW, EPT)
    srcp = jnp.pad(src, ((0, 0), (0, RPAD - EPT)))
    dstp = jnp.pad(dst, ((0, 0), (0, RPAD - EPT)))
    sd64 = jnp.concatenate([srcp.reshape(NW, NB2, 1, BE),
                            dstp.reshape(NW, NB2, 1, BE)], axis=2)

    p, dpart = _edge_kernel(s, sd64)
    hbf = jnp.pad(h_input.astype(jnp.bfloat16), ((0, RPAD - N), (0, 0)))
    acc = _msg_kernel(dpart, p, sd64, hbf)
    return _blend(embedding_input, acc)


# speedup vs baseline: 1.0053x; 1.0053x over previous
import functools

import jax
import jax.numpy as jnp
from jax import lax
from jax.experimental import pallas as pl
from jax.experimental.pallas import tpu as pltpu
from jax.experimental.pallas import tpu_sc as plsc

N = 10000
D = 128
E = 320000
ALPHA = 0.7
BETA = 0.3
GAMMA = 0.2

NC = 2
NS = 16
NW = NC * NS
EPT = E // NW
ROWS = 80
RPAD = ROWS * 128
NB2X = 160

_mesh = plsc.VectorSubcoreMesh(core_axis_name="c", subcore_axis_name="s")


def _scores_body(wa_ref, wfc_ref, h_ref, out_ref, hbf_ref):
    A = jnp.dot(wa_ref[...], wfc_ref[...], preferred_element_type=jnp.float32)
    s = lax.dot_general(
        A, h_ref[...], (((1,), (1,)), ((), ())),
        preferred_element_type=jnp.float32)
    out_ref[...] = jnp.pad(s, ((0, 0), (0, RPAD - N)))
    hbf_ref[...] = jnp.pad(h_ref[...].astype(jnp.bfloat16),
                           ((0, RPAD - N), (0, 0)))


def _scores(wa, wfc, h):
    return pl.pallas_call(
        _scores_body,
        out_shape=(jax.ShapeDtypeStruct((2, RPAD), jnp.float32),
                   jax.ShapeDtypeStruct((RPAD, D), jnp.bfloat16)),
    )(wa, wfc, h)


def _edge_body(s_hbm, sd_hbm, p_hbm, dpart_hbm,
               s1_t, s2_t, sd_t, p_t, zb_t, dsum, sem):
    cid = lax.axis_index("c")
    sid = lax.axis_index("s")
    wid = sid * NC + cid

    pltpu.sync_copy(s_hbm.at[0], s1_t)
    pltpu.sync_copy(s_hbm.at[1], s2_t)
    pltpu.sync_copy(sd_hbm.at[wid], sd_t)

    for c in range(128):
        zb_t[pl.ds(c * 16, 16)] = jnp.zeros((16,), jnp.float32)

    @pl.when(sid == 0)
    def _():
        for k in range(RPAD // 2048):
            pltpu.sync_copy(zb_t, dsum.at[pl.ds(k * 2048, 2048)])

    plsc.subcore_barrier()

    iota16 = lax.broadcasted_iota(jnp.int32, (16,), 0)

    def row(r, _):
        def col(c, _):
            srcv = sd_t[r, 0, pl.ds(c * 16, 16)]
            dstv = sd_t[r, 1, pl.ds(c * 16, 16)]
            s1v = plsc.load_gather(s1_t, [srcv])
            s2v = plsc.load_gather(s2_t, [dstv])
            a = s1v + s2v
            e = jnp.where(a > 0, a, GAMMA * a)
            e = jnp.minimum(e, 60.0)
            p = jnp.exp(e)
            lid = r * 64 + c * 16 + iota16
            p = jnp.where(lid < EPT, p, 0.0)
            p_t[r, 0, pl.ds(c * 16, 16)] = p
            return 0
        lax.fori_loop(0, 4, col, 0)
        return 0
    lax.fori_loop(0, NB2X, row, 0)

    def srow(r, _):
        pltpu.sync_copy(p_t.at[r, 0], dsum.at[sd_t.at[r, 1]], add=True)
        return 0
    lax.fori_loop(0, NB2X, srow, 0)

    plsc.subcore_barrier()

    pltpu.sync_copy(p_t, p_hbm.at[wid])

    @pl.when(sid == 0)
    def _():
        pltpu.sync_copy(dsum, dpart_hbm.at[cid])


@functools.partial(
    pl.kernel,
    out_type=(jax.ShapeDtypeStruct((NW, NB2X, 1, 64), jnp.float32),
              jax.ShapeDtypeStruct((NC, RPAD), jnp.float32)),
    mesh=_mesh,
    compiler_params=pltpu.CompilerParams(needs_layout_passes=False),
    scratch_types=[
        pltpu.VMEM((RPAD,), jnp.float32),
        pltpu.VMEM((RPAD,), jnp.float32),
        pltpu.VMEM((NB2X, 2, 64), jnp.int32),
        pltpu.VMEM((NB2X, 1, 64), jnp.float32),
        pltpu.VMEM((2048,), jnp.float32),
        pltpu.VMEM_SHARED((RPAD,), jnp.float32),
        pltpu.SemaphoreType.DMA,
    ],
)
def _edge_kernel(s_hbm, sd_hbm, p_hbm, dpart_hbm, *scratch):
    _edge_body(s_hbm, sd_hbm, p_hbm, dpart_hbm, *scratch)


def _dmerge_body(d_ref, out_ref):
    out_ref[...] = d_ref[0] + d_ref[1]


def _dmerge(dpart):
    return pl.pallas_call(
        _dmerge_body,
        out_shape=jax.ShapeDtypeStruct((ROWS, 128), jnp.float32),
    )(dpart.reshape(NC, ROWS, 128))


_DNUMS = lax.GatherDimensionNumbers(
    offset_dims=(), collapsed_slice_dims=(0,), start_index_map=(0,))
NB2 = 160
BE = 64


def _msg_body(denom_hbm, p_hbm, sd_hbm, h_hbm, acc_hbm,
              denom_t, sd0, sd1, sd2, sd3, sd4, sd5, sd6, sd7,
              pr0, pr1, pr2, pr3, pr4, pr5, pr6, pr7,
              rb0, rb1, rb2, rb3, hsp, acc,
              si0, si1, si2, si3, si4, si5, si6, si7,
              sg0, sg1, sg2, sg3, ss0, ss1, ss2, ss3):
    cid = lax.axis_index("c")
    sid = lax.axis_index("s")
    wid = sid * NC + cid
    sd = (sd0, sd1, sd2, sd3, sd4, sd5, sd6, sd7)
    pr = (pr0, pr1, pr2, pr3, pr4, pr5, pr6, pr7)
    si = (si0, si1, si2, si3, si4, si5, si6, si7)
    rb = (rb0, rb1, rb2, rb3)
    sg = (sg0, sg1, sg2, sg3)
    ss = (ss0, ss1, ss2, ss3)
    base = sid * 640

    pltpu.sync_copy(denom_hbm, denom_t)

    def zrow(i, _):
        def zcol(c, _):
            rb0[i, pl.ds(c * 32, 32)] = jnp.zeros((32,), jnp.bfloat16)
            return 0
        lax.fori_loop(0, 4, zcol, 0)
        return 0
    lax.fori_loop(0, BE, zrow, 0)
    for k in range(10):
        pltpu.sync_copy(rb0, acc.at[pl.ds(base + k * BE, BE)])

    for k in range(10):
        pltpu.sync_copy(h_hbm.at[pl.ds(base + k * BE, BE)], rb1)
        pltpu.sync_copy(rb1, hsp.at[pl.ds(base + k * BE, BE)])

    plsc.subcore_barrier()

    def start_idx(r, s):
        pltpu.async_copy(sd_hbm.at[wid, r], sd[s], si[s])
        pltpu.async_copy(p_hbm.at[wid, r], pr[s], si[s])

    def wait_idx(r, s):
        pltpu.make_async_copy(sd_hbm.at[wid, r], sd[s], si[s]).wait()
        pltpu.make_async_copy(p_hbm.at[wid, r], pr[s], si[s]).wait()

    def start_gather(s, b):
        pltpu.async_copy(hsp.at[sd[s].at[0]], rb[b], sg[b])

    def wait_gather(b):
        pltpu.make_async_copy(hsp.at[sd[0].at[0]], rb[b], sg[b]).wait()

    def start_scatter(s, b):
        pltpu.async_copy(rb[b], acc.at[sd[s].at[1]], ss[b], add=True)

    def wait_scatter(b):
        pltpu.make_async_copy(rb[b], acc.at[sd[0].at[1]], ss[b]).wait()

    start_idx(0, 0)
    start_idx(1, 1)
    start_idx(2, 2)
    wait_idx(0, 0)
    start_gather(0, 0)
    wait_idx(1, 1)
    start_gather(1, 1)

    def oct_(i, _):
        for k in range(8):
            r = i * 8 + k
            b = k % 4
            s = k

            @pl.when(r + 3 < NB2)
            def _():
                start_idx(r + 3, (k + 3) % 8)

            @pl.when(r + 2 < NB2)
            def _():
                @pl.when(r >= 2)
                def _():
                    wait_scatter((k + 2) % 4)
                wait_idx(r + 2, (k + 2) % 8)
                start_gather((k + 2) % 8, (k + 2) % 4)

            wait_gather(b)

            def scale(c, _):
                dstv = sd[s][1, pl.ds(c * 16, 16)]
                dv = plsc.load_gather(denom_t, [dstv])
                pv = pr[s][0, pl.ds(c * 16, 16)]
                attnv = pv / (dv + 1e-16)
                for j in range(16):
                    sp = lax.gather(
                        attnv, jnp.full((16, 1), j, jnp.int32), _DNUMS, (1,),
                        mode=lax.GatherScatterMode.PROMISE_IN_BOUNDS)
                    spb = plsc.pack(sp, sp, format=plsc.PackFormat.INTERLEAVED)
                    row = c * 16 + j
                    for dch in range(4):
                        rb[b][row, pl.ds(dch * 32, 32)] = (
                            rb[b][row, pl.ds(dch * 32, 32)] * spb)
                return 0
            lax.fori_loop(0, BE // 16, scale, 0)

            start_scatter(s, b)
        return 0
    lax.fori_loop(0, NB2 // 8, oct_, 0)

    for b in range(4):
        wait_scatter(b)

    plsc.subcore_barrier()

    for k in range(5):
        pltpu.sync_copy(acc.at[pl.ds(base + k * 128, 128)],
                        acc_hbm.at[cid, pl.ds(base + k * 128, 128)])


@functools.partial(
    pl.kernel,
    out_type=jax.ShapeDtypeStruct((NC, RPAD, D), jnp.bfloat16),
    mesh=_mesh,
    compiler_params=pltpu.CompilerParams(
        needs_layout_passes=False, use_tc_tiling_on_sc=False),
    scratch_types=(
        [pltpu.VMEM((RPAD,), jnp.float32)]
        + [pltpu.VMEM((2, BE), jnp.int32)] * 8
        + [pltpu.VMEM((1, BE), jnp.float32)] * 8
        + [pltpu.VMEM((BE, D), jnp.bfloat16)] * 4
        + [pltpu.VMEM_SHARED((RPAD, D), jnp.bfloat16)]
        + [pltpu.VMEM_SHARED((RPAD, D), jnp.bfloat16)]
        + [pltpu.SemaphoreType.DMA] * 16
    ),
)
def _msg_kernel(denom_hbm, p_hbm, sd_hbm, h_hbm, acc_hbm, *scratch):
    _msg_body(denom_hbm, p_hbm, sd_hbm, h_hbm, acc_hbm, *scratch)


def _blend_body(emb_ref, acc_ref, out_ref):
    crf = (acc_ref[0].astype(jnp.float32) + acc_ref[1].astype(jnp.float32))
    out_ref[...] = (ALPHA * emb_ref[...] + BETA * crf) / (ALPHA + BETA)


def _blend(emb, acc):
    blk = 2000
    return pl.pallas_call(
        _blend_body,
        grid=(N // blk,),
        in_specs=[pl.BlockSpec((blk, D), lambda g: (g, 0)),
                  pl.BlockSpec((NC, blk, D), lambda g: (0, g, 0))],
        out_specs=pl.BlockSpec((blk, D), lambda g: (g, 0)),
        out_shape=jax.ShapeDtypeStruct((N, D), jnp.float32),
    )(emb, acc)


def kernel(embedding_input, h_input, edge_index, W_fc, W_attn):
    wa = W_attn.reshape(2, D)
    s, hbf = _scores(wa, W_fc, h_input)

    src = edge_index[0].reshape(NW, EPT)
    dst = edge_index[1].reshape(NW, EPT)
    srcp = jnp.pad(src, ((0, 0), (0, RPAD - EPT)))
    dstp = jnp.pad(dst, ((0, 0), (0, RPAD - EPT)))
    sd64 = jnp.concatenate([srcp.reshape(NW, NB2, 1, BE),
                            dstp.reshape(NW, NB2, 1, BE)], axis=2)

    p, dpart = _edge_kernel(s, sd64)
    denom = _dmerge(dpart).reshape(RPAD)
    acc = _msg_kernel(denom, p, sd64, hbf)
    return _blend(embedding_input, acc)


# speedup vs baseline: 1.0117x; 1.0064x over previous
import functools

import jax
import jax.numpy as jnp
from jax import lax
from jax.experimental import pallas as pl
from jax.experimental.pallas import tpu as pltpu
from jax.experimental.pallas import tpu_sc as plsc

N = 10000
D = 128
E = 320000
ALPHA = 0.7
BETA = 0.3
GAMMA = 0.2

NC = 2
NS = 16
NW = NC * NS
EPT = E // NW
ROWS = 80
RPAD = ROWS * 128

_mesh = plsc.VectorSubcoreMesh(core_axis_name="c", subcore_axis_name="s")


def _scores_body(wa_ref, wfc_ref, h_ref, out_ref, hbf_ref):
    A = jnp.dot(wa_ref[...], wfc_ref[...], preferred_element_type=jnp.float32)
    s = lax.dot_general(
        A, h_ref[...], (((1,), (1,)), ((), ())),
        preferred_element_type=jnp.float32)
    out_ref[...] = jnp.pad(s, ((0, 0), (0, RPAD - N)))
    hbf_ref[...] = jnp.pad(h_ref[...].astype(jnp.bfloat16),
                           ((0, RPAD - N), (0, 0)))


def _scores(wa, wfc, h):
    return pl.pallas_call(
        _scores_body,
        out_shape=(jax.ShapeDtypeStruct((2, RPAD), jnp.float32),
                   jax.ShapeDtypeStruct((RPAD, D), jnp.bfloat16)),
    )(wa, wfc, h)


def _edge_body(s_hbm, sd_hbm, p_hbm, dpart_hbm,
               s1_t, s2_t, sd_t, p_t, zb_t, dsum, sem):
    cid = lax.axis_index("c")
    sid = lax.axis_index("s")
    wid = sid * NC + cid

    pltpu.sync_copy(s_hbm.at[0], s1_t)
    pltpu.sync_copy(s_hbm.at[1], s2_t)
    pltpu.sync_copy(sd_hbm.at[wid], sd_t)

    for c in range(128):
        zb_t[pl.ds(c * 16, 16)] = jnp.zeros((16,), jnp.float32)

    @pl.when(sid == 0)
    def _():
        for k in range(RPAD // 2048):
            pltpu.sync_copy(zb_t, dsum.at[pl.ds(k * 2048, 2048)])

    plsc.subcore_barrier()

    iota16 = lax.broadcasted_iota(jnp.int32, (16,), 0)

    def row(r, _):
        def col(c, _):
            srcv = sd_t[r, 0, pl.ds(c * 16, 16)]
            dstv = sd_t[r, 1, pl.ds(c * 16, 16)]
            s1v = plsc.load_gather(s1_t, [srcv])
            s2v = plsc.load_gather(s2_t, [dstv])
            a = s1v + s2v
            e = jnp.where(a > 0, a, GAMMA * a)
            e = jnp.minimum(e, 60.0)
            p = jnp.exp(e)
            lid = r * 128 + c * 16 + iota16
            p = jnp.where(lid < EPT, p, 0.0)
            p_t[r, 0, pl.ds(c * 16, 16)] = p
            return 0
        lax.fori_loop(0, 8, col, 0)
        return 0
    lax.fori_loop(0, ROWS, row, 0)

    def srow(r, _):
        pltpu.sync_copy(p_t.at[r, 0], dsum.at[sd_t.at[r, 1]], add=True)
        return 0
    lax.fori_loop(0, ROWS, srow, 0)

    plsc.subcore_barrier()

    pltpu.sync_copy(p_t, p_hbm.at[wid])

    @pl.when(sid == 0)
    def _():
        pltpu.sync_copy(dsum, dpart_hbm.at[cid])


@functools.partial(
    pl.kernel,
    out_type=(jax.ShapeDtypeStruct((NW, ROWS, 1, 128), jnp.float32),
              jax.ShapeDtypeStruct((NC, RPAD), jnp.float32)),
    mesh=_mesh,
    compiler_params=pltpu.CompilerParams(needs_layout_passes=False),
    scratch_types=[
        pltpu.VMEM((RPAD,), jnp.float32),
        pltpu.VMEM((RPAD,), jnp.float32),
        pltpu.VMEM((ROWS, 2, 128), jnp.int32),
        pltpu.VMEM((ROWS, 1, 128), jnp.float32),
        pltpu.VMEM((2048,), jnp.float32),
        pltpu.VMEM_SHARED((RPAD,), jnp.float32),
        pltpu.SemaphoreType.DMA,
    ],
)
def _edge_kernel(s_hbm, sd_hbm, p_hbm, dpart_hbm, *scratch):
    _edge_body(s_hbm, sd_hbm, p_hbm, dpart_hbm, *scratch)


def _dmerge_body(d_ref, out_ref):
    out_ref[...] = d_ref[0] + d_ref[1]


def _dmerge(dpart):
    return pl.pallas_call(
        _dmerge_body,
        out_shape=jax.ShapeDtypeStruct((ROWS, 128), jnp.float32),
    )(dpart.reshape(NC, ROWS, 128))


_DNUMS = lax.GatherDimensionNumbers(
    offset_dims=(), collapsed_slice_dims=(0,), start_index_map=(0,))
NB2 = 160
BE = 64


def _msg_body(denom_hbm, p_hbm, sd_hbm, h_hbm, acc_hbm,
              denom_t, sd0, sd1, sd2, sd3, sd4, sd5, sd6, sd7,
              pr0, pr1, pr2, pr3, pr4, pr5, pr6, pr7,
              rb0, rb1, rb2, rb3, hsp, acc,
              si0, si1, si2, si3, si4, si5, si6, si7,
              sg0, sg1, sg2, sg3, ss0, ss1, ss2, ss3):
    cid = lax.axis_index("c")
    sid = lax.axis_index("s")
    wid = sid * NC + cid
    sd = (sd0, sd1, sd2, sd3, sd4, sd5, sd6, sd7)
    pr = (pr0, pr1, pr2, pr3, pr4, pr5, pr6, pr7)
    si = (si0, si1, si2, si3, si4, si5, si6, si7)
    rb = (rb0, rb1, rb2, rb3)
    sg = (sg0, sg1, sg2, sg3)
    ss = (ss0, ss1, ss2, ss3)
    base = sid * 640

    pltpu.sync_copy(denom_hbm, denom_t)

    def zrow(i, _):
        def zcol(c, _):
            rb0[i, pl.ds(c * 32, 32)] = jnp.zeros((32,), jnp.bfloat16)
            return 0
        lax.fori_loop(0, 4, zcol, 0)
        return 0
    lax.fori_loop(0, BE, zrow, 0)
    for k in range(10):
        pltpu.sync_copy(rb0, acc.at[pl.ds(base + k * BE, BE)])

    for k in range(10):
        pltpu.sync_copy(h_hbm.at[pl.ds(base + k * BE, BE)], rb1)
        pltpu.sync_copy(rb1, hsp.at[pl.ds(base + k * BE, BE)])

    plsc.subcore_barrier()

    def start_idx(r, s):
        pltpu.async_copy(sd_hbm.at[wid, r], sd[s], si[s])
        pltpu.async_copy(p_hbm.at[wid, r], pr[s], si[s])

    def wait_idx(r, s):
        pltpu.make_async_copy(sd_hbm.at[wid, r], sd[s], si[s]).wait()
        pltpu.make_async_copy(p_hbm.at[wid, r], pr[s], si[s]).wait()

    def start_gather(s, b):
        pltpu.async_copy(hsp.at[sd[s].at[0]], rb[b], sg[b])

    def wait_gather(b):
        pltpu.make_async_copy(hsp.at[sd[0].at[0]], rb[b], sg[b]).wait()

    def start_scatter(s, b):
        pltpu.async_copy(rb[b], acc.at[sd[s].at[1]], ss[b], add=True)

    def wait_scatter(b):
        pltpu.make_async_copy(rb[b], acc.at[sd[0].at[1]], ss[b]).wait()

    start_idx(0, 0)
    start_idx(1, 1)
    start_idx(2, 2)
    wait_idx(0, 0)
    start_gather(0, 0)
    wait_idx(1, 1)
    start_gather(1, 1)

    def oct_(i, _):
        for k in range(8):
            r = i * 8 + k
            b = k % 4
            s = k

            @pl.when(r + 3 < NB2)
            def _():
                start_idx(r + 3, (k + 3) % 8)

            @pl.when(r + 2 < NB2)
            def _():
                @pl.when(r >= 2)
                def _():
                    wait_scatter((k + 2) % 4)
                wait_idx(r + 2, (k + 2) % 8)
                start_gather((k + 2) % 8, (k + 2) % 4)

            wait_gather(b)

            def scale(c, _):
                dstv = sd[s][1, pl.ds(c * 16, 16)]
                dv = plsc.load_gather(denom_t, [dstv])
                pv = pr[s][0, pl.ds(c * 16, 16)]
                attnv = pv / (dv + 1e-16)
                for j in range(16):
                    sp = lax.gather(
                        attnv, jnp.full((16, 1), j, jnp.int32), _DNUMS, (1,),
                        mode=lax.GatherScatterMode.PROMISE_IN_BOUNDS)
                    spb = plsc.pack(sp, sp, format=plsc.PackFormat.INTERLEAVED)
                    row = c * 16 + j
                    for dch in range(4):
                        rb[b][row, pl.ds(dch * 32, 32)] = (
                            rb[b][row, pl.ds(dch * 32, 32)] * spb)
                return 0
            lax.fori_loop(0, BE // 16, scale, 0)

            start_scatter(s, b)
        return 0
    lax.fori_loop(0, NB2 // 8, oct_, 0)

    for b in range(4):
        wait_scatter(b)

    plsc.subcore_barrier()

    for k in range(5):
        pltpu.sync_copy(acc.at[pl.ds(base + k * 128, 128)],
                        acc_hbm.at[cid, pl.ds(base + k * 128, 128)])


@functools.partial(
    pl.kernel,
    out_type=jax.ShapeDtypeStruct((NC, RPAD, D), jnp.bfloat16),
    mesh=_mesh,
    compiler_params=pltpu.CompilerParams(
        needs_layout_passes=False, use_tc_tiling_on_sc=False),
    scratch_types=(
        [pltpu.VMEM((RPAD,), jnp.float32)]
        + [pltpu.VMEM((2, BE), jnp.int32)] * 8
        + [pltpu.VMEM((1, BE), jnp.float32)] * 8
        + [pltpu.VMEM((BE, D), jnp.bfloat16)] * 4
        + [pltpu.VMEM_SHARED((RPAD, D), jnp.bfloat16)]
        + [pltpu.VMEM_SHARED((RPAD, D), jnp.bfloat16)]
        + [pltpu.SemaphoreType.DMA] * 16
    ),
)
def _msg_kernel(denom_hbm, p_hbm, sd_hbm, h_hbm, acc_hbm, *scratch):
    _msg_body(denom_hbm, p_hbm, sd_hbm, h_hbm, acc_hbm, *scratch)


def _blend_body(emb_ref, acc_ref, out_ref):
    crf = (acc_ref[0].astype(jnp.float32) + acc_ref[1].astype(jnp.float32))
    out_ref[...] = (ALPHA * emb_ref[...] + BETA * crf) / (ALPHA + BETA)


def _blend(emb, acc):
    blk = 2000
    return pl.pallas_call(
        _blend_body,
        grid=(N // blk,),
        in_specs=[pl.BlockSpec((blk, D), lambda g: (g, 0)),
                  pl.BlockSpec((NC, blk, D), lambda g: (0, g, 0))],
        out_specs=pl.BlockSpec((blk, D), lambda g: (g, 0)),
        out_shape=jax.ShapeDtypeStruct((N, D), jnp.float32),
    )(emb, acc)


def kernel(embedding_input, h_input, edge_index, W_fc, W_attn):
    wa = W_attn.reshape(2, D)
    s, hbf = _scores(wa, W_fc, h_input)

    src = edge_index[0].reshape(NW, EPT)
    dst = edge_index[1].reshape(NW, EPT)
    srcp = jnp.pad(src, ((0, 0), (0, RPAD - EPT)))
    dstp = jnp.pad(dst, ((0, 0), (0, RPAD - EPT)))
    sd = jnp.concatenate([srcp.reshape(NW, ROWS, 1, 128),
                          dstp.reshape(NW, ROWS, 1, 128)], axis=2)
    sd64 = jnp.concatenate([srcp.reshape(NW, NB2, 1, BE),
                            dstp.reshape(NW, NB2, 1, BE)], axis=2)

    p, dpart = _edge_kernel(s, sd)
    denom = _dmerge(dpart).reshape(RPAD)
    acc = _msg_kernel(denom, p.reshape(NW, NB2, 1, BE), sd64, hbf)
    return _blend(embedding_input, acc)
